# baseline (speedup 1.0000x reference)
import functools

import jax
import jax.numpy as jnp
from jax import lax
from jax.experimental import pallas as pl
from jax.experimental.pallas import tpu as pltpu
from jax.experimental.pallas import tpu_sc as plsc

PREFIX_LENGTH = 128
NUM_LAYERS = 24
HIDDEN_SIZE = 1024
BATCH = 16
EMBED_DIM = NUM_LAYERS * HIDDEN_SIZE
B = BATCH * PREFIX_LENGTH
V = PREFIX_LENGTH

NC, NS = 2, 16
HALF = EMBED_DIM // NC
RPT = B // NS
VPT = V // NS
NVEC = RPT // 16

_mesh = plsc.VectorSubcoreMesh(core_axis_name="c", subcore_axis_name="s")


@functools.partial(
    pl.kernel,
    mesh=_mesh,
    out_type=jax.ShapeDtypeStruct((B, EMBED_DIM), jnp.float32),
    scratch_types=[
        pltpu.VMEM((RPT,), jnp.int32),
        pltpu.VMEM_SHARED((V, HALF), jnp.float32),
        pltpu.SemaphoreType.DMA,
        pltpu.SemaphoreType.DMA,
    ],
)
def _gather_kernel(idx_hbm, table_hbm, out_hbm, idx_v, shared_tab,
                   sem0, sem1):
    c = lax.axis_index("c")
    s = lax.axis_index("s")
    sems = (sem0, sem1)
    col0 = c * HALF

    pltpu.sync_copy(
        table_hbm.at[pl.ds(VPT * s, VPT), pl.ds(col0, HALF)],
        shared_tab.at[pl.ds(VPT * s, VPT)],
    )
    pltpu.sync_copy(idx_hbm.at[s], idx_v)
    plsc.subcore_barrier()

    row_base = s * RPT
    out_at = lambda k: out_hbm.at[pl.ds(row_base + k, 1), pl.ds(col0, HALF)]

    def body(j2, carry):
        for half in range(2):
            j = 2 * j2 + half
            vecs = idx_v[pl.ds(j * 16, 16)]
            for lane in range(16):
                pltpu.async_copy(
                    shared_tab.at[pl.ds(vecs[lane], 1)],
                    out_at(j * 16 + lane),
                    sems[half],
                )

            def _drain_prev(half=half, off=-16):
                for lane in range(16):
                    pltpu.make_async_copy(
                        shared_tab.at[pl.ds(0, 1)],
                        out_at(j * 16 + off + lane),
                        sems[1 - half],
                    ).wait()

            if half == 1:
                _drain_prev()
            else:
                pl.when(j2 > 0)(_drain_prev)
        return carry

    lax.fori_loop(0, NVEC // 2, body, 0)
    for lane in range(16):
        pltpu.make_async_copy(
            shared_tab.at[pl.ds(0, 1)],
            out_at((NVEC - 1) * 16 + lane),
            sems[1],
        ).wait()


def kernel(prefix, table):
    idx = prefix.astype(jnp.int32).reshape(NS, RPT)
    out = _gather_kernel(idx, table)
    return out.reshape(BATCH, PREFIX_LENGTH, EMBED_DIM)


# speedup vs baseline: 1.1580x; 1.1580x over previous
import functools

import jax
import jax.numpy as jnp
from jax import lax
from jax.experimental import pallas as pl
from jax.experimental.pallas import tpu as pltpu
from jax.experimental.pallas import tpu_sc as plsc

PREFIX_LENGTH = 128
NUM_LAYERS = 24
HIDDEN_SIZE = 1024
BATCH = 16
EMBED_DIM = NUM_LAYERS * HIDDEN_SIZE
B = BATCH * PREFIX_LENGTH
V = PREFIX_LENGTH

NC, NS = 2, 16
NPHASE = 2
Q = EMBED_DIM // (NC * NPHASE)
RPT = B // NS
VPT = V // NS
NBUF = 8
DIST = 6
NVEC = RPT // 16

_mesh = plsc.VectorSubcoreMesh(core_axis_name="c", subcore_axis_name="s")


@functools.partial(
    pl.kernel,
    mesh=_mesh,
    out_type=jax.ShapeDtypeStruct((B, EMBED_DIM), jnp.float32),
    scratch_types=(
        [pltpu.VMEM((RPT,), jnp.int32)]
        + [pltpu.VMEM((1, Q), jnp.float32) for _ in range(NBUF)]
        + [pltpu.VMEM_SHARED((V, Q), jnp.float32)]
        + [pltpu.SemaphoreType.DMA for _ in range(2 * NBUF)]
    ),
)
def _gather_kernel(idx_hbm, table_hbm, out_hbm, idx_v, *rest):
    bufs = rest[:NBUF]
    shared_tab = rest[NBUF]
    gsem = rest[NBUF + 1:NBUF + 1 + NBUF]
    ssem = rest[NBUF + 1 + NBUF:]
    c = lax.axis_index("c")
    s = lax.axis_index("s")

    pltpu.sync_copy(idx_hbm.at[s], idx_v)
    row_base = s * RPT

    for p in range(NPHASE):
        col0 = c * (NPHASE * Q) + p * Q
        out_at = lambda k: out_hbm.at[pl.ds(row_base + k, 1), pl.ds(col0, Q)]

        def gather(v, b, from_hbm, col0=col0):
            if from_hbm:
                src = table_hbm.at[pl.ds(v, 1), pl.ds(col0, Q)]
            else:
                src = shared_tab.at[pl.ds(v, 1)]
            pltpu.async_copy(src, bufs[b], gsem[b])

        if p > 0:
            plsc.subcore_barrier()
        pltpu.sync_copy(
            table_hbm.at[pl.ds(VPT * s, VPT), pl.ds(col0, Q)],
            shared_tab.at[pl.ds(VPT * s, VPT)],
        )
        plsc.subcore_barrier()

        vec0 = idx_v[pl.ds(0, 16)]
        for k in range(DIST):
            gather(vec0[k], k % NBUF, k % 8 == 3)

        def body(j, carry):
            vecs = idx_v[pl.ds(j * 16, 16)]
            vecs2 = idx_v[pl.ds(lax.min(j * 16 + 16, RPT - 16), 16)]

            def _wait_store(b2, koff):
                pltpu.make_async_copy(
                    bufs[b2], out_at(j * 16 + koff), ssem[b2]
                ).wait()

            for lane in range(16):
                b = lane % NBUF
                k = j * 16 + lane

                pltpu.make_async_copy(
                    shared_tab.at[pl.ds(0, 1)], bufs[b], gsem[b]
                ).wait()
                pltpu.async_copy(bufs[b], out_at(k), ssem[b])

                b2 = (lane + DIST) % NBUF
                if lane + DIST < 16:
                    if lane + DIST >= NBUF:
                        _wait_store(b2, lane + DIST - NBUF)
                    else:
                        pl.when(j > 0)(
                            functools.partial(
                                _wait_store, b2, lane + DIST - NBUF))
                    gather(vecs[lane + DIST], b2, (lane + DIST) % 8 == 3)
                else:
                    @pl.when(j < NVEC - 1)
                    def _prefetch_next_group(lane=lane, b2=b2):
                        _wait_store(b2, lane + DIST - NBUF)
                        gather(vecs2[lane + DIST - 16], b2,
                               (lane + DIST) % 8 == 3)
            return carry

        lax.fori_loop(0, NVEC, body, 0)
        for b in range(NBUF):
            pltpu.make_async_copy(
                bufs[b], out_at(RPT - NBUF + b), ssem[b]
            ).wait()


def kernel(prefix, table):
    idx = prefix.astype(jnp.int32).reshape(NS, RPT)
    out = _gather_kernel(idx, table)
    return out.reshape(BATCH, PREFIX_LENGTH, EMBED_DIM)


# speedup vs baseline: 1.1796x; 1.0187x over previous
import functools

import jax
import jax.numpy as jnp
from jax import lax
from jax.experimental import pallas as pl
from jax.experimental.pallas import tpu as pltpu
from jax.experimental.pallas import tpu_sc as plsc

PREFIX_LENGTH = 128
NUM_LAYERS = 24
HIDDEN_SIZE = 1024
BATCH = 16
EMBED_DIM = NUM_LAYERS * HIDDEN_SIZE
B = BATCH * PREFIX_LENGTH
V = PREFIX_LENGTH

NC, NS = 2, 16
NPHASE = 2
Q = EMBED_DIM // (NC * NPHASE)
RPT = B // NS
VPT = V // NS
NBUF = 4
PDIST = 2
NVEC = RPT // 16
PPG = 8

_mesh = plsc.VectorSubcoreMesh(core_axis_name="c", subcore_axis_name="s")


@functools.partial(
    pl.kernel,
    mesh=_mesh,
    out_type=jax.ShapeDtypeStruct((B, EMBED_DIM), jnp.float32),
    scratch_types=(
        [pltpu.VMEM((RPT,), jnp.int32)]
        + [pltpu.VMEM((2, Q), jnp.float32) for _ in range(NBUF)]
        + [pltpu.VMEM_SHARED((V, Q), jnp.float32)]
        + [pltpu.SemaphoreType.DMA for _ in range(2 * NBUF)]
    ),
)
def _gather_kernel(idx_hbm, table_hbm, out_hbm, idx_v, *rest):
    bufs = rest[:NBUF]
    shared_tab = rest[NBUF]
    gsem = rest[NBUF + 1:NBUF + 1 + NBUF]
    ssem = rest[NBUF + 1 + NBUF:]
    c = lax.axis_index("c")
    s = lax.axis_index("s")

    pltpu.sync_copy(idx_hbm.at[s], idx_v)
    row_base = s * RPT

    def gather(v, b, h):
        pltpu.async_copy(
            shared_tab.at[pl.ds(v, 1)], bufs[b].at[pl.ds(h, 1)], gsem[b]
        )

    for p in range(NPHASE):
        col0 = c * (NPHASE * Q) + p * Q
        out_at = lambda P: out_hbm.at[
            pl.ds(row_base + 2 * P, 2), pl.ds(col0, Q)]

        if p > 0:
            plsc.subcore_barrier()
        pltpu.sync_copy(
            table_hbm.at[pl.ds(VPT * s, VPT), pl.ds(col0, Q)],
            shared_tab.at[pl.ds(VPT * s, VPT)],
        )
        plsc.subcore_barrier()

        vec0 = idx_v[pl.ds(0, 16)]
        for P in range(PDIST):
            gather(vec0[2 * P], P % NBUF, 0)
            gather(vec0[2 * P + 1], P % NBUF, 1)

        def body(j, carry):
            vecs = idx_v[pl.ds(j * 16, 16)]
            vecs2 = idx_v[pl.ds(lax.min(j * 16 + 16, RPT - 16), 16)]

            def _wait_store(b2, poff):
                pltpu.make_async_copy(
                    bufs[b2], out_at(j * PPG + poff), ssem[b2]
                ).wait()

            for q in range(PPG):
                b = q % NBUF
                P = j * PPG + q

                pltpu.make_async_copy(
                    shared_tab.at[pl.ds(0, 2)], bufs[b], gsem[b]
                ).wait()
                pltpu.async_copy(bufs[b], out_at(P), ssem[b])

                b2 = (q + PDIST) % NBUF
                if q + PDIST < PPG:
                    if q + PDIST >= NBUF:
                        _wait_store(b2, q + PDIST - NBUF)
                    else:
                        pl.when(j > 0)(
                            functools.partial(
                                _wait_store, b2, q + PDIST - NBUF))
                    gather(vecs[2 * (q + PDIST)], b2, 0)
                    gather(vecs[2 * (q + PDIST) + 1], b2, 1)
                else:
                    @pl.when(j < NVEC - 1)
                    def _prefetch_next_group(q=q, b2=b2):
                        _wait_store(b2, q + PDIST - NBUF)
                        gather(vecs2[2 * (q + PDIST - PPG)], b2, 0)
                        gather(vecs2[2 * (q + PDIST - PPG) + 1], b2, 1)
            return carry

        lax.fori_loop(0, NVEC, body, 0)
        for b in range(NBUF):
            pltpu.make_async_copy(
                bufs[b], out_at(RPT // 2 - NBUF + b), ssem[b]
            ).wait()


def kernel(prefix, table):
    idx = prefix.astype(jnp.int32).reshape(NS, RPT)
    out = _gather_kernel(idx, table)
    return out.reshape(BATCH, PREFIX_LENGTH, EMBED_DIM)


# speedup vs baseline: 1.3476x; 1.1425x over previous
import functools

import jax
import jax.numpy as jnp
from jax import lax
from jax.experimental import pallas as pl
from jax.experimental.pallas import tpu as pltpu
from jax.experimental.pallas import tpu_sc as plsc

PREFIX_LENGTH = 128
NUM_LAYERS = 24
HIDDEN_SIZE = 1024
BATCH = 16
EMBED_DIM = NUM_LAYERS * HIDDEN_SIZE
B = BATCH * PREFIX_LENGTH
V = PREFIX_LENGTH

NC, NS = 2, 16
NPHASE = 2
Q = EMBED_DIM // (NC * NPHASE)
RPT = B // NS
VPT = V // NS
NVEC = RPT // 16

LP = [l for l in range(16) if l % 4 != 3]
LD = [l for l in range(16) if l % 4 == 3]
NP = len(LP)
NBUF = 6
DIST = 4
ND = len(LD)

_mesh = plsc.VectorSubcoreMesh(core_axis_name="c", subcore_axis_name="s")


@functools.partial(
    pl.kernel,
    mesh=_mesh,
    out_type=jax.ShapeDtypeStruct((B, EMBED_DIM), jnp.float32),
    scratch_types=(
        [pltpu.VMEM((RPT,), jnp.int32)]
        + [pltpu.VMEM((1, Q), jnp.float32) for _ in range(NBUF)]
        + [pltpu.VMEM_SHARED((V, Q), jnp.float32)]
        + [pltpu.SemaphoreType.DMA for _ in range(2 * NBUF + ND)]
    ),
)
def _gather_kernel(idx_hbm, table_hbm, out_hbm, idx_v, *rest):
    bufs = rest[:NBUF]
    shared_tab = rest[NBUF]
    gsem = rest[NBUF + 1:NBUF + 1 + NBUF]
    ssem = rest[NBUF + 1 + NBUF:NBUF + 1 + 2 * NBUF]
    dsem = rest[NBUF + 1 + 2 * NBUF:]
    c = lax.axis_index("c")
    s = lax.axis_index("s")

    pltpu.sync_copy(idx_hbm.at[s], idx_v)
    row_base = s * RPT

    def gather(v, b):
        pltpu.async_copy(shared_tab.at[pl.ds(v, 1)], bufs[b], gsem[b])

    for p in range(NPHASE):
        col0 = c * (NPHASE * Q) + p * Q
        out_at = lambda k: out_hbm.at[pl.ds(row_base + k, 1), pl.ds(col0, Q)]

        if p > 0:
            plsc.subcore_barrier()
        pltpu.sync_copy(
            table_hbm.at[pl.ds(VPT * s, VPT), pl.ds(col0, Q)],
            shared_tab.at[pl.ds(VPT * s, VPT)],
        )
        plsc.subcore_barrier()

        vec0 = idx_v[pl.ds(0, 16)]
        for o in range(DIST):
            gather(vec0[LP[o]], o % NBUF)

        def body(j, carry):
            vecs = idx_v[pl.ds(j * 16, 16)]
            vecs2 = idx_v[pl.ds(lax.min(j * 16 + 16, RPT - 16), 16)]

            def _wait_store(b2, krel):
                pltpu.make_async_copy(
                    bufs[b2], out_at(j * 16 + krel), ssem[b2]
                ).wait()

            for od, ld in enumerate(LD):
                def _wait_prev_direct(od=od, ld=ld):
                    pltpu.make_async_copy(
                        shared_tab.at[pl.ds(0, 1)],
                        out_at((j - 1) * 16 + ld), dsem[od]
                    ).wait()
                pl.when(j > 0)(_wait_prev_direct)
                pltpu.async_copy(
                    shared_tab.at[pl.ds(vecs[ld], 1)],
                    out_at(j * 16 + ld), dsem[od])

            for o in range(NP):
                b = o % NBUF
                k = j * 16 + LP[o]

                pltpu.make_async_copy(
                    shared_tab.at[pl.ds(0, 1)], bufs[b], gsem[b]
                ).wait()
                pltpu.async_copy(bufs[b], out_at(k), ssem[b])

                b2 = (o + DIST) % NBUF
                if o + DIST < NP:
                    if o + DIST >= NBUF:
                        _wait_store(b2, LP[o + DIST - NBUF])
                    else:
                        pl.when(j > 0)(
                            functools.partial(
                                _wait_store, b2, LP[o + DIST - NBUF] - 16))
                    gather(vecs[LP[o + DIST]], b2)
                else:
                    @pl.when(j < NVEC - 1)
                    def _prefetch_next_group(o=o, b2=b2):
                        _wait_store(b2, LP[o + DIST - NBUF])
                        gather(vecs2[LP[o + DIST - NP]], b2)
            return carry

        lax.fori_loop(0, NVEC, body, 0)
        for o in range(NP - NBUF, NP):
            pltpu.make_async_copy(
                bufs[o % NBUF], out_at((NVEC - 1) * 16 + LP[o]),
                ssem[o % NBUF]
            ).wait()
        for od, ld in enumerate(LD):
            pltpu.make_async_copy(
                shared_tab.at[pl.ds(0, 1)],
                out_at((NVEC - 1) * 16 + ld), dsem[od]
            ).wait()


def kernel(prefix, table):
    idx = prefix.astype(jnp.int32).reshape(NS, RPT)
    out = _gather_kernel(idx, table)
    return out.reshape(BATCH, PREFIX_LENGTH, EMBED_DIM)


# speedup vs baseline: 1.3590x; 1.0084x over previous
import functools

import jax
import jax.numpy as jnp
from jax import lax
from jax.experimental import pallas as pl
from jax.experimental.pallas import tpu as pltpu
from jax.experimental.pallas import tpu_sc as plsc

PREFIX_LENGTH = 128
NUM_LAYERS = 24
HIDDEN_SIZE = 1024
BATCH = 16
EMBED_DIM = NUM_LAYERS * HIDDEN_SIZE
B = BATCH * PREFIX_LENGTH
V = PREFIX_LENGTH

NC, NS = 2, 16
NPHASE = 2
Q = EMBED_DIM // (NC * NPHASE)
RPT = B // NS
VPT = V // NS
NVEC = RPT // 16

LP = [l for l in range(16) if l % 8 < 5]
LD = [l for l in range(16) if l % 8 >= 5]
NP = len(LP)
NBUF = 5
DIST = 3
ND = len(LD)

_mesh = plsc.VectorSubcoreMesh(core_axis_name="c", subcore_axis_name="s")


@functools.partial(
    pl.kernel,
    mesh=_mesh,
    out_type=jax.ShapeDtypeStruct((B, EMBED_DIM), jnp.float32),
    scratch_types=(
        [pltpu.VMEM((RPT,), jnp.int32)]
        + [pltpu.VMEM((1, Q), jnp.float32) for _ in range(NBUF)]
        + [pltpu.VMEM_SHARED((V, Q), jnp.float32)]
        + [pltpu.SemaphoreType.DMA for _ in range(2 * NBUF + ND)]
    ),
)
def _gather_kernel(idx_hbm, table_hbm, out_hbm, idx_v, *rest):
    bufs = rest[:NBUF]
    shared_tab = rest[NBUF]
    gsem = rest[NBUF + 1:NBUF + 1 + NBUF]
    ssem = rest[NBUF + 1 + NBUF:NBUF + 1 + 2 * NBUF]
    dsem = rest[NBUF + 1 + 2 * NBUF:]
    c = lax.axis_index("c")
    s = lax.axis_index("s")

    pltpu.sync_copy(idx_hbm.at[s], idx_v)
    row_base = s * RPT

    def gather(v, b):
        pltpu.async_copy(shared_tab.at[pl.ds(v, 1)], bufs[b], gsem[b])

    for p in range(NPHASE):
        col0 = c * (NPHASE * Q) + p * Q
        out_at = lambda k: out_hbm.at[pl.ds(row_base + k, 1), pl.ds(col0, Q)]

        if p > 0:
            plsc.subcore_barrier()
        pltpu.sync_copy(
            table_hbm.at[pl.ds(VPT * s, VPT), pl.ds(col0, Q)],
            shared_tab.at[pl.ds(VPT * s, VPT)],
        )
        plsc.subcore_barrier()

        vec0 = idx_v[pl.ds(0, 16)]
        for o in range(DIST):
            gather(vec0[LP[o]], o % NBUF)

        def body(j, carry):
            vecs = idx_v[pl.ds(j * 16, 16)]
            vecs2 = idx_v[pl.ds(lax.min(j * 16 + 16, RPT - 16), 16)]

            def _wait_store(b2, krel):
                pltpu.make_async_copy(
                    bufs[b2], out_at(j * 16 + krel), ssem[b2]
                ).wait()

            for od, ld in enumerate(LD):
                def _wait_prev_direct(od=od, ld=ld):
                    pltpu.make_async_copy(
                        shared_tab.at[pl.ds(0, 1)],
                        out_at((j - 1) * 16 + ld), dsem[od]
                    ).wait()
                pl.when(j > 0)(_wait_prev_direct)
                pltpu.async_copy(
                    shared_tab.at[pl.ds(vecs[ld], 1)],
                    out_at(j * 16 + ld), dsem[od])

            for o in range(NP):
                b = o % NBUF
                k = j * 16 + LP[o]

                pltpu.make_async_copy(
                    shared_tab.at[pl.ds(0, 1)], bufs[b], gsem[b]
                ).wait()
                pltpu.async_copy(bufs[b], out_at(k), ssem[b])

                b2 = (o + DIST) % NBUF
                if o + DIST < NP:
                    if o + DIST >= NBUF:
                        _wait_store(b2, LP[o + DIST - NBUF])
                    else:
                        pl.when(j > 0)(
                            functools.partial(
                                _wait_store, b2, LP[o + DIST - NBUF] - 16))
                    gather(vecs[LP[o + DIST]], b2)
                else:
                    @pl.when(j < NVEC - 1)
                    def _prefetch_next_group(o=o, b2=b2):
                        _wait_store(b2, LP[o + DIST - NBUF])
                        gather(vecs2[LP[o + DIST - NP]], b2)
            return carry

        lax.fori_loop(0, NVEC, body, 0)
        for o in range(NP - NBUF, NP):
            pltpu.make_async_copy(
                bufs[o % NBUF], out_at((NVEC - 1) * 16 + LP[o]),
                ssem[o % NBUF]
            ).wait()
        for od, ld in enumerate(LD):
            pltpu.make_async_copy(
                shared_tab.at[pl.ds(0, 1)],
                out_at((NVEC - 1) * 16 + ld), dsem[od]
            ).wait()


def kernel(prefix, table):
    idx = prefix.astype(jnp.int32).reshape(NS, RPT)
    out = _gather_kernel(idx, table)
    return out.reshape(BATCH, PREFIX_LENGTH, EMBED_DIM)
